# baseline (speedup 1.0000x reference)
import functools

import jax
import jax.numpy as jnp
from jax import lax
from jax.experimental import pallas as pl
from jax.experimental.pallas import tpu as pltpu
from jax.experimental.pallas import tpu_sc as plsc

_N = 10000
_E = 320000
_D = 128
_DH = _D // 2
_BN_EPS = 1e-5

_NC = 2
_NS = 16
_K = 128
_EPAD = 327680
_SPT = _EPAD // (_NS * _K)
_NP = 10240
_RPT = _NP // _NS
_G = 4
_GROUPS = _SPT // _G

_EPW = _EPAD // (_NC * _NS)
_DSTEPS = _EPW // _K
_DGROUPS = _DSTEPS // _G


def _sc_mesh():
    return plsc.VectorSubcoreMesh(
        core_axis_name="c", subcore_axis_name="s",
        num_cores=_NC, num_subcores=_NS,
    )


@functools.cache
def _get_sc_deg():
    @functools.partial(
        pl.kernel,
        out_type=jax.ShapeDtypeStruct((_NC, _NP, 16), jnp.float32),
        mesh=_sc_mesh(),
        scratch_types=[
            pltpu.VMEM((_DSTEPS, _K), jnp.int32),
            pltpu.VMEM((_K, 16), jnp.float32),
            pltpu.VMEM_SHARED((_NP, 16), jnp.float32),
        ] + [pltpu.SemaphoreType.DMA] * _G,
        compiler_params=pltpu.CompilerParams(use_tc_tiling_on_sc=False),
    )
    def _sc_deg(dst_hbm, zrows_hbm, ones_hbm, out_hbm, dstall, onesb, acc,
                *sems):
        c = lax.axis_index("c")
        s = lax.axis_index("s")
        wid = c * _NS + s
        pltpu.sync_copy(zrows_hbm, acc.at[pl.ds(s * _RPT, _RPT)])
        pltpu.sync_copy(ones_hbm, onesb)
        pltpu.sync_copy(dst_hbm.at[wid], dstall)
        plsc.subcore_barrier()

        def grp(g, carry):
            for b in range(_G):
                @pl.when(g > 0)
                def _():
                    pltpu.make_async_copy(zrows_hbm.at[pl.ds(0, _K)], onesb,
                                          sems[b]).wait()
                pltpu.async_copy(onesb, acc.at[dstall.at[g * _G + b]],
                                 sems[b], add=True)
            return carry

        lax.fori_loop(0, _DGROUPS, grp, 0)
        for b in range(_G):
            pltpu.make_async_copy(zrows_hbm.at[pl.ds(0, _K)], onesb,
                                  sems[b]).wait()
        plsc.subcore_barrier()
        pltpu.sync_copy(acc.at[pl.ds(s * _RPT, _RPT)],
                        out_hbm.at[c, pl.ds(s * _RPT, _RPT)])

    return _sc_deg


@functools.cache
def _get_sc_agg():
    @functools.partial(
        pl.kernel,
        out_type=jax.ShapeDtypeStruct((_NC, _NP, _DH), jnp.float32),
        mesh=_sc_mesh(),
        scratch_types=[
            pltpu.VMEM((_SPT, _K), jnp.int32),
            pltpu.VMEM((_SPT, _K), jnp.int32),
            pltpu.VMEM((_G, _K, _DH), jnp.float32),
            pltpu.VMEM_SHARED((_NP, _DH), jnp.float32),
        ] + [pltpu.SemaphoreType.DMA] * (2 * _G),
        compiler_params=pltpu.CompilerParams(use_tc_tiling_on_sc=False),
    )
    def _sc_agg(y_hbm, src_hbm, dst_hbm, out_hbm, srcall, dstall, rows, acc,
                *sems):
        sg = sems[:_G]
        ss = sems[_G:]
        c = lax.axis_index("c")
        s = lax.axis_index("s")
        pltpu.sync_copy(y_hbm.at[c].at[pl.ds(s * _RPT, _RPT)],
                        acc.at[pl.ds(s * _RPT, _RPT)])
        pltpu.sync_copy(src_hbm.at[s], srcall)
        pltpu.sync_copy(dst_hbm.at[s], dstall)
        plsc.subcore_barrier()
        table = y_hbm.at[c]

        def grp(g, carry):
            gd = []
            for b in range(_G):
                @pl.when(g > 0)
                def _():
                    pltpu.make_async_copy(y_hbm.at[0].at[pl.ds(0, _K)],
                                          rows.at[b], ss[b]).wait()
                gd.append(pltpu.async_copy(table.at[srcall.at[g * _G + b]],
                                           rows.at[b], sg[b]))
            for b in range(_G):
                gd[b].wait()
                pltpu.async_copy(rows.at[b], acc.at[dstall.at[g * _G + b]],
                                 ss[b], add=True)
            return carry

        lax.fori_loop(0, _GROUPS, grp, 0)
        for b in range(_G):
            pltpu.make_async_copy(y_hbm.at[0].at[pl.ds(0, _K)], rows.at[b],
                                  ss[b]).wait()
        plsc.subcore_barrier()
        pltpu.sync_copy(acc.at[pl.ds(s * _RPT, _RPT)],
                        out_hbm.at[c, pl.ds(s * _RPT, _RPT)])

    return _sc_agg


_BR = 1024


def _dinv_block(degp_ref):
    deg = degp_ref[0, :, 0:1] + degp_ref[1, :, 0:1]
    return lax.rsqrt(1.0 + deg)


def _split_out(out_ref, y):
    out_ref[0] = y[:, :_DH]
    out_ref[1] = y[:, _DH:]


def _unsplit(p_ref):
    return jnp.concatenate([p_ref[0], p_ref[1]], axis=1)


def _tc_first_body(x_ref, degp_ref, w_ref, s_ref, out_ref):
    dinv = _dinv_block(degp_ref)
    xw = jnp.dot(x_ref[...], w_ref[...], preferred_element_type=jnp.float32)
    _split_out(out_ref, dinv * (xw * s_ref[...][None, :]))


def _tc_mid_body(p_ref, degp_ref, cv_ref, wn_ref, sn_ref, out_ref):
    dinv = _dinv_block(degp_ref)
    agg = _unsplit(p_ref)
    a = jnp.maximum(dinv * agg + cv_ref[...][None, :], 0.0)
    aw = jnp.dot(a, wn_ref[...], preferred_element_type=jnp.float32)
    _split_out(out_ref, dinv * (aw * sn_ref[...][None, :]))


def _tc_last_body(p_ref, degp_ref, b_ref, out_ref):
    dinv = _dinv_block(degp_ref)
    agg = _unsplit(p_ref)
    out_ref[...] = dinv * agg + b_ref[...][None, :]


_row_spec = pl.BlockSpec((_BR, _D), lambda i: (i, 0))
_half_spec = pl.BlockSpec((_NC, _BR, _DH), lambda i: (0, i, 0))
_degp_spec = pl.BlockSpec((_NC, _BR, 16), lambda i: (0, i, 0))
_vec_spec = pl.BlockSpec((_D,), lambda i: (0,))
_mat_spec = pl.BlockSpec((_D, _D), lambda i: (0, 0))
_half_sds = jax.ShapeDtypeStruct((_NC, _NP, _DH), jnp.float32)
_full_sds = jax.ShapeDtypeStruct((_NP, _D), jnp.float32)
_grid = (_NP // _BR,)

_tc_first = pl.pallas_call(
    _tc_first_body, grid=_grid, out_shape=_half_sds,
    in_specs=[_row_spec, _degp_spec, _mat_spec, _vec_spec],
    out_specs=_half_spec,
)
_tc_mid = pl.pallas_call(
    _tc_mid_body, grid=_grid, out_shape=_half_sds,
    in_specs=[_half_spec, _degp_spec, _vec_spec, _mat_spec, _vec_spec],
    out_specs=_half_spec,
)
_tc_last = pl.pallas_call(
    _tc_last_body, grid=_grid, out_shape=_full_sds,
    in_specs=[_half_spec, _degp_spec, _vec_spec],
    out_specs=_row_spec,
)


@jax.jit
def kernel(x, edge_index, W1, b1, g1, bt1, W2, b2, g2, bt2, W3, b3):
    pad_idx = _N + (jnp.arange(_EPAD - _E, dtype=jnp.int32) % (_NP - _N))
    srcf = jnp.concatenate([edge_index[0], pad_idx])
    dstf = jnp.concatenate([edge_index[1], pad_idx])
    src = srcf.reshape(_NS, _SPT, _K)
    dst = dstf.reshape(_NS, _SPT, _K)
    dst_deg = dstf.reshape(_NC * _NS, _DSTEPS, _K)
    xp = jnp.concatenate([x, jnp.zeros((_NP - _N, _D), jnp.float32)])

    isq = (1.0 + _BN_EPS) ** -0.5
    s1 = g1 * isq
    c1 = s1 * b1 + bt1
    s2 = g2 * isq
    c2 = s2 * b2 + bt2
    ones_d = jnp.ones((_D,), jnp.float32)
    zrows = jnp.zeros((_RPT, 16), jnp.float32)
    ones_k = jnp.ones((_K, 16), jnp.float32)

    sc_deg = _get_sc_deg()
    sc_agg = _get_sc_agg()
    degp = sc_deg(dst_deg, zrows, ones_k)
    y1 = _tc_first(xp, degp, W1, s1)
    p1 = sc_agg(y1, src, dst)
    y2 = _tc_mid(p1, degp, c1, W2, s2)
    p2 = sc_agg(y2, src, dst)
    y3 = _tc_mid(p2, degp, c2, W3, ones_d)
    p3 = sc_agg(y3, src, dst)
    return _tc_last(p3, degp, b3)[:_N]


# speedup vs baseline: 1.0338x; 1.0338x over previous
import functools

import jax
import jax.numpy as jnp
from jax import lax
from jax.experimental import pallas as pl
from jax.experimental.pallas import tpu as pltpu
from jax.experimental.pallas import tpu_sc as plsc

_N = 10000
_E = 320000
_D = 128
_DH = _D // 2
_BN_EPS = 1e-5

_NC = 2
_NS = 16
_K = 128
_EPAD = 327680
_SPT = _EPAD // (_NS * _K)
_NP = 10240
_RPT = _NP // _NS
_G = 8
_GROUPS = _SPT // _G

_EPW = _EPAD // (_NC * _NS)
_DSTEPS = _EPW // _K
_DGROUPS = _DSTEPS // _G


def _sc_mesh():
    return plsc.VectorSubcoreMesh(
        core_axis_name="c", subcore_axis_name="s",
        num_cores=_NC, num_subcores=_NS,
    )


@functools.cache
def _get_sc_deg():
    @functools.partial(
        pl.kernel,
        out_type=jax.ShapeDtypeStruct((_NC, _NP, 16), jnp.float32),
        mesh=_sc_mesh(),
        scratch_types=[
            pltpu.VMEM((_DSTEPS, _K), jnp.int32),
            pltpu.VMEM((_K, 16), jnp.float32),
            pltpu.VMEM_SHARED((_NP, 16), jnp.float32),
        ] + [pltpu.SemaphoreType.DMA] * _G,
        compiler_params=pltpu.CompilerParams(use_tc_tiling_on_sc=False),
    )
    def _sc_deg(dst_hbm, zrows_hbm, ones_hbm, out_hbm, dstall, onesb, acc,
                *sems):
        c = lax.axis_index("c")
        s = lax.axis_index("s")
        wid = c * _NS + s
        pltpu.sync_copy(zrows_hbm, acc.at[pl.ds(s * _RPT, _RPT)])
        pltpu.sync_copy(ones_hbm, onesb)
        pltpu.sync_copy(dst_hbm.at[wid], dstall)
        plsc.subcore_barrier()

        def grp(g, carry):
            for b in range(_G):
                @pl.when(g > 0)
                def _():
                    pltpu.make_async_copy(zrows_hbm.at[pl.ds(0, _K)], onesb,
                                          sems[b]).wait()
                pltpu.async_copy(onesb, acc.at[dstall.at[g * _G + b]],
                                 sems[b], add=True)
            return carry

        lax.fori_loop(0, _DGROUPS, grp, 0)
        for b in range(_G):
            pltpu.make_async_copy(zrows_hbm.at[pl.ds(0, _K)], onesb,
                                  sems[b]).wait()
        plsc.subcore_barrier()
        pltpu.sync_copy(acc.at[pl.ds(s * _RPT, _RPT)],
                        out_hbm.at[c, pl.ds(s * _RPT, _RPT)])

    return _sc_deg


@functools.cache
def _get_sc_agg():
    @functools.partial(
        pl.kernel,
        out_type=jax.ShapeDtypeStruct((_NC, _NP, _DH), jnp.float32),
        mesh=_sc_mesh(),
        scratch_types=[
            pltpu.VMEM((2, _G, _K), jnp.int32),
            pltpu.VMEM((2, _G, _K), jnp.int32),
            pltpu.VMEM((_G, _K, _DH), jnp.float32),
            pltpu.VMEM_SHARED((_NP, _DH), jnp.float32),
        ] + [pltpu.SemaphoreType.DMA] * (2 * _G + 1),
        compiler_params=pltpu.CompilerParams(use_tc_tiling_on_sc=False),
    )
    def _sc_agg(y_hbm, src_hbm, dst_hbm, out_hbm, srcb, dstb, rows, acc,
                *sems):
        sg = sems[:_G]
        ss = sems[_G:2 * _G]
        si = sems[2 * _G]
        c = lax.axis_index("c")
        s = lax.axis_index("s")
        pltpu.sync_copy(y_hbm.at[c].at[pl.ds(s * _RPT, _RPT)],
                        acc.at[pl.ds(s * _RPT, _RPT)])
        pltpu.sync_copy(src_hbm.at[s, pl.ds(0, _G)], srcb.at[0])
        pltpu.sync_copy(dst_hbm.at[s, pl.ds(0, _G)], dstb.at[0])
        plsc.subcore_barrier()
        table = y_hbm.at[c]

        def grp(g, carry):
            p = lax.rem(g, 2)
            @pl.when(g > 0)
            def _():
                pltpu.make_async_copy(src_hbm.at[s, pl.ds(0, _G)],
                                      srcb.at[p], si).wait()
                pltpu.make_async_copy(dst_hbm.at[s, pl.ds(0, _G)],
                                      dstb.at[p], si).wait()
            gd = []
            for b in range(_G):
                @pl.when(g > 0)
                def _():
                    pltpu.make_async_copy(y_hbm.at[0].at[pl.ds(0, _K)],
                                          rows.at[b], ss[b]).wait()
                gd.append(pltpu.async_copy(table.at[srcb.at[p, b]],
                                           rows.at[b], sg[b]))
            @pl.when(g + 1 < _GROUPS)
            def _():
                pltpu.async_copy(src_hbm.at[s, pl.ds((g + 1) * _G, _G)],
                                 srcb.at[1 - p], si)
                pltpu.async_copy(dst_hbm.at[s, pl.ds((g + 1) * _G, _G)],
                                 dstb.at[1 - p], si)
            for b in range(_G):
                gd[b].wait()
                pltpu.async_copy(rows.at[b], acc.at[dstb.at[p, b]],
                                 ss[b], add=True)
            return carry

        lax.fori_loop(0, _GROUPS, grp, 0)
        for b in range(_G):
            pltpu.make_async_copy(y_hbm.at[0].at[pl.ds(0, _K)], rows.at[b],
                                  ss[b]).wait()
        plsc.subcore_barrier()
        pltpu.sync_copy(acc.at[pl.ds(s * _RPT, _RPT)],
                        out_hbm.at[c, pl.ds(s * _RPT, _RPT)])

    return _sc_agg


_BR = 1024


def _dinv_block(degp_ref):
    deg = degp_ref[0, :, 0:1] + degp_ref[1, :, 0:1]
    return lax.rsqrt(1.0 + deg)


def _split_out(out_ref, y):
    out_ref[0] = y[:, :_DH]
    out_ref[1] = y[:, _DH:]


def _unsplit(p_ref):
    return jnp.concatenate([p_ref[0], p_ref[1]], axis=1)


def _tc_first_body(x_ref, degp_ref, w_ref, s_ref, out_ref):
    dinv = _dinv_block(degp_ref)
    xw = jnp.dot(x_ref[...], w_ref[...], preferred_element_type=jnp.float32)
    _split_out(out_ref, dinv * (xw * s_ref[...][None, :]))


def _tc_mid_body(p_ref, degp_ref, cv_ref, wn_ref, sn_ref, out_ref):
    dinv = _dinv_block(degp_ref)
    agg = _unsplit(p_ref)
    a = jnp.maximum(dinv * agg + cv_ref[...][None, :], 0.0)
    aw = jnp.dot(a, wn_ref[...], preferred_element_type=jnp.float32)
    _split_out(out_ref, dinv * (aw * sn_ref[...][None, :]))


def _tc_last_body(p_ref, degp_ref, b_ref, out_ref):
    dinv = _dinv_block(degp_ref)
    agg = _unsplit(p_ref)
    out_ref[...] = dinv * agg + b_ref[...][None, :]


_row_spec = pl.BlockSpec((_BR, _D), lambda i: (i, 0))
_half_spec = pl.BlockSpec((_NC, _BR, _DH), lambda i: (0, i, 0))
_degp_spec = pl.BlockSpec((_NC, _BR, 16), lambda i: (0, i, 0))
_vec_spec = pl.BlockSpec((_D,), lambda i: (0,))
_mat_spec = pl.BlockSpec((_D, _D), lambda i: (0, 0))
_half_sds = jax.ShapeDtypeStruct((_NC, _NP, _DH), jnp.float32)
_full_sds = jax.ShapeDtypeStruct((_NP, _D), jnp.float32)
_grid = (_NP // _BR,)

_tc_first = pl.pallas_call(
    _tc_first_body, grid=_grid, out_shape=_half_sds,
    in_specs=[_row_spec, _degp_spec, _mat_spec, _vec_spec],
    out_specs=_half_spec,
)
_tc_mid = pl.pallas_call(
    _tc_mid_body, grid=_grid, out_shape=_half_sds,
    in_specs=[_half_spec, _degp_spec, _vec_spec, _mat_spec, _vec_spec],
    out_specs=_half_spec,
)
_tc_last = pl.pallas_call(
    _tc_last_body, grid=_grid, out_shape=_full_sds,
    in_specs=[_half_spec, _degp_spec, _vec_spec],
    out_specs=_row_spec,
)


@jax.jit
def kernel(x, edge_index, W1, b1, g1, bt1, W2, b2, g2, bt2, W3, b3):
    pad_idx = _N + (jnp.arange(_EPAD - _E, dtype=jnp.int32) % (_NP - _N))
    srcf = jnp.concatenate([edge_index[0], pad_idx])
    dstf = jnp.concatenate([edge_index[1], pad_idx])
    src = srcf.reshape(_NS, _SPT, _K)
    dst = dstf.reshape(_NS, _SPT, _K)
    dst_deg = dstf.reshape(_NC * _NS, _DSTEPS, _K)
    xp = jnp.concatenate([x, jnp.zeros((_NP - _N, _D), jnp.float32)])

    isq = (1.0 + _BN_EPS) ** -0.5
    s1 = g1 * isq
    c1 = s1 * b1 + bt1
    s2 = g2 * isq
    c2 = s2 * b2 + bt2
    ones_d = jnp.ones((_D,), jnp.float32)
    zrows = jnp.zeros((_RPT, 16), jnp.float32)
    ones_k = jnp.ones((_K, 16), jnp.float32)

    sc_deg = _get_sc_deg()
    sc_agg = _get_sc_agg()
    degp = sc_deg(dst_deg, zrows, ones_k)
    y1 = _tc_first(xp, degp, W1, s1)
    p1 = sc_agg(y1, src, dst)
    y2 = _tc_mid(p1, degp, c1, W2, s2)
    p2 = sc_agg(y2, src, dst)
    y3 = _tc_mid(p2, degp, c2, W3, ones_d)
    p3 = sc_agg(y3, src, dst)
    return _tc_last(p3, degp, b3)[:_N]


# speedup vs baseline: 1.0368x; 1.0029x over previous
import functools

import jax
import jax.numpy as jnp
from jax import lax
from jax.experimental import pallas as pl
from jax.experimental.pallas import tpu as pltpu
from jax.experimental.pallas import tpu_sc as plsc

_N = 10000
_E = 320000
_D = 128
_DH = _D // 2
_BN_EPS = 1e-5

_NC = 2
_NS = 16
_K = 128
_EPAD = 327680
_SPT = _EPAD // (_NS * _K)
_NP = 10240
_RPT = _NP // _NS
_G = 10
_GROUPS = _SPT // _G

_EPW = _EPAD // (_NC * _NS)
_DSTEPS = _EPW // _K
_DGROUPS = _DSTEPS // _G


def _sc_mesh():
    return plsc.VectorSubcoreMesh(
        core_axis_name="c", subcore_axis_name="s",
        num_cores=_NC, num_subcores=_NS,
    )


@functools.cache
def _get_sc_deg():
    @functools.partial(
        pl.kernel,
        out_type=jax.ShapeDtypeStruct((_NC, _NP, 16), jnp.float32),
        mesh=_sc_mesh(),
        scratch_types=[
            pltpu.VMEM((_DSTEPS, _K), jnp.int32),
            pltpu.VMEM((_K, 16), jnp.float32),
            pltpu.VMEM_SHARED((_NP, 16), jnp.float32),
        ] + [pltpu.SemaphoreType.DMA] * _G,
        compiler_params=pltpu.CompilerParams(use_tc_tiling_on_sc=False),
    )
    def _sc_deg(dst_hbm, zrows_hbm, ones_hbm, out_hbm, dstall, onesb, acc,
                *sems):
        c = lax.axis_index("c")
        s = lax.axis_index("s")
        wid = c * _NS + s
        pltpu.sync_copy(zrows_hbm, acc.at[pl.ds(s * _RPT, _RPT)])
        pltpu.sync_copy(ones_hbm, onesb)
        pltpu.sync_copy(dst_hbm.at[wid], dstall)
        plsc.subcore_barrier()

        def grp(g, carry):
            for b in range(_G):
                @pl.when(g > 0)
                def _():
                    pltpu.make_async_copy(zrows_hbm.at[pl.ds(0, _K)], onesb,
                                          sems[b]).wait()
                pltpu.async_copy(onesb, acc.at[dstall.at[g * _G + b]],
                                 sems[b], add=True)
            return carry

        lax.fori_loop(0, _DGROUPS, grp, 0)
        for b in range(_G):
            pltpu.make_async_copy(zrows_hbm.at[pl.ds(0, _K)], onesb,
                                  sems[b]).wait()
        plsc.subcore_barrier()
        pltpu.sync_copy(acc.at[pl.ds(s * _RPT, _RPT)],
                        out_hbm.at[c, pl.ds(s * _RPT, _RPT)])

    return _sc_deg


@functools.cache
def _get_sc_agg():
    @functools.partial(
        pl.kernel,
        out_type=jax.ShapeDtypeStruct((_NC, _NP, _DH), jnp.float32),
        mesh=_sc_mesh(),
        scratch_types=[
            pltpu.VMEM((2, _G, _K), jnp.int32),
            pltpu.VMEM((2, _G, _K), jnp.int32),
            pltpu.VMEM((_G, _K, _DH), jnp.float32),
            pltpu.VMEM_SHARED((_NP, _DH), jnp.float32),
        ] + [pltpu.SemaphoreType.DMA] * (2 * _G + 1),
        compiler_params=pltpu.CompilerParams(use_tc_tiling_on_sc=False),
    )
    def _sc_agg(y_hbm, src_hbm, dst_hbm, out_hbm, srcb, dstb, rows, acc,
                *sems):
        sg = sems[:_G]
        ss = sems[_G:2 * _G]
        si = sems[2 * _G]
        c = lax.axis_index("c")
        s = lax.axis_index("s")
        pltpu.sync_copy(y_hbm.at[c].at[pl.ds(s * _RPT, _RPT)],
                        acc.at[pl.ds(s * _RPT, _RPT)])
        pltpu.sync_copy(src_hbm.at[s, pl.ds(0, _G)], srcb.at[0])
        pltpu.sync_copy(dst_hbm.at[s, pl.ds(0, _G)], dstb.at[0])
        plsc.subcore_barrier()
        table = y_hbm.at[c]

        def grp(g, carry):
            p = lax.rem(g, 2)
            @pl.when(g > 0)
            def _():
                pltpu.make_async_copy(src_hbm.at[s, pl.ds(0, _G)],
                                      srcb.at[p], si).wait()
                pltpu.make_async_copy(dst_hbm.at[s, pl.ds(0, _G)],
                                      dstb.at[p], si).wait()
            gd = []
            for b in range(_G):
                @pl.when(g > 0)
                def _():
                    pltpu.make_async_copy(y_hbm.at[0].at[pl.ds(0, _K)],
                                          rows.at[b], ss[b]).wait()
                gd.append(pltpu.async_copy(table.at[srcb.at[p, b]],
                                           rows.at[b], sg[b]))
            @pl.when(g + 1 < _GROUPS)
            def _():
                pltpu.async_copy(src_hbm.at[s, pl.ds((g + 1) * _G, _G)],
                                 srcb.at[1 - p], si)
                pltpu.async_copy(dst_hbm.at[s, pl.ds((g + 1) * _G, _G)],
                                 dstb.at[1 - p], si)
            for b in range(_G):
                gd[b].wait()
                pltpu.async_copy(rows.at[b], acc.at[dstb.at[p, b]],
                                 ss[b], add=True)
            return carry

        lax.fori_loop(0, _GROUPS, grp, 0)
        for b in range(_G):
            pltpu.make_async_copy(y_hbm.at[0].at[pl.ds(0, _K)], rows.at[b],
                                  ss[b]).wait()
        plsc.subcore_barrier()
        pltpu.sync_copy(acc.at[pl.ds(s * _RPT, _RPT)],
                        out_hbm.at[c, pl.ds(s * _RPT, _RPT)])

    return _sc_agg


_BR = 1024


def _dinv_block(degp_ref):
    deg = degp_ref[0, :, 0:1] + degp_ref[1, :, 0:1]
    return lax.rsqrt(1.0 + deg)


def _split_out(out_ref, y):
    out_ref[0] = y[:, :_DH]
    out_ref[1] = y[:, _DH:]


def _unsplit(p_ref):
    return jnp.concatenate([p_ref[0], p_ref[1]], axis=1)


def _tc_first_body(x_ref, degp_ref, w_ref, s_ref, out_ref):
    dinv = _dinv_block(degp_ref)
    xw = jnp.dot(x_ref[...], w_ref[...], preferred_element_type=jnp.float32)
    _split_out(out_ref, dinv * (xw * s_ref[...][None, :]))


def _tc_mid_body(p_ref, degp_ref, cv_ref, wn_ref, sn_ref, out_ref):
    dinv = _dinv_block(degp_ref)
    agg = _unsplit(p_ref)
    a = jnp.maximum(dinv * agg + cv_ref[...][None, :], 0.0)
    aw = jnp.dot(a, wn_ref[...], preferred_element_type=jnp.float32)
    _split_out(out_ref, dinv * (aw * sn_ref[...][None, :]))


def _tc_last_body(p_ref, degp_ref, b_ref, out_ref):
    dinv = _dinv_block(degp_ref)
    agg = _unsplit(p_ref)
    out_ref[...] = dinv * agg + b_ref[...][None, :]


_row_spec = pl.BlockSpec((_BR, _D), lambda i: (i, 0))
_half_spec = pl.BlockSpec((_NC, _BR, _DH), lambda i: (0, i, 0))
_degp_spec = pl.BlockSpec((_NC, _BR, 16), lambda i: (0, i, 0))
_vec_spec = pl.BlockSpec((_D,), lambda i: (0,))
_mat_spec = pl.BlockSpec((_D, _D), lambda i: (0, 0))
_half_sds = jax.ShapeDtypeStruct((_NC, _NP, _DH), jnp.float32)
_full_sds = jax.ShapeDtypeStruct((_NP, _D), jnp.float32)
_grid = (_NP // _BR,)

_tc_first = pl.pallas_call(
    _tc_first_body, grid=_grid, out_shape=_half_sds,
    in_specs=[_row_spec, _degp_spec, _mat_spec, _vec_spec],
    out_specs=_half_spec,
)
_tc_mid = pl.pallas_call(
    _tc_mid_body, grid=_grid, out_shape=_half_sds,
    in_specs=[_half_spec, _degp_spec, _vec_spec, _mat_spec, _vec_spec],
    out_specs=_half_spec,
)
_tc_last = pl.pallas_call(
    _tc_last_body, grid=_grid, out_shape=_full_sds,
    in_specs=[_half_spec, _degp_spec, _vec_spec],
    out_specs=_row_spec,
)


@jax.jit
def kernel(x, edge_index, W1, b1, g1, bt1, W2, b2, g2, bt2, W3, b3):
    pad_idx = _N + (jnp.arange(_EPAD - _E, dtype=jnp.int32) % (_NP - _N))
    srcf = jnp.concatenate([edge_index[0], pad_idx])
    dstf = jnp.concatenate([edge_index[1], pad_idx])
    src = srcf.reshape(_NS, _SPT, _K)
    dst = dstf.reshape(_NS, _SPT, _K)
    dst_deg = dstf.reshape(_NC * _NS, _DSTEPS, _K)
    xp = jnp.concatenate([x, jnp.zeros((_NP - _N, _D), jnp.float32)])

    isq = (1.0 + _BN_EPS) ** -0.5
    s1 = g1 * isq
    c1 = s1 * b1 + bt1
    s2 = g2 * isq
    c2 = s2 * b2 + bt2
    ones_d = jnp.ones((_D,), jnp.float32)
    zrows = jnp.zeros((_RPT, 16), jnp.float32)
    ones_k = jnp.ones((_K, 16), jnp.float32)

    sc_deg = _get_sc_deg()
    sc_agg = _get_sc_agg()
    degp = sc_deg(dst_deg, zrows, ones_k)
    y1 = _tc_first(xp, degp, W1, s1)
    p1 = sc_agg(y1, src, dst)
    y2 = _tc_mid(p1, degp, c1, W2, s2)
    p2 = sc_agg(y2, src, dst)
    y3 = _tc_mid(p2, degp, c2, W3, ones_d)
    p3 = sc_agg(y3, src, dst)
    return _tc_last(p3, degp, b3)[:_N]
